# baseline (speedup 1.0000x reference)
import dataclasses
import functools

import jax
import jax.numpy as jnp
from jax.experimental import pallas as pl
from jax.experimental.pallas import tpu as pltpu
from jax.experimental.pallas import tpu_sc as plsc

_EMA = 0.99
_THRESHOLD = 0.7


def _dot(a, b, dims):
    return jax.lax.dot_general(a, b, (dims, ((), ())),
                               precision=jax.lax.Precision.DEFAULT,
                               preferred_element_type=jnp.float32)


def _gelu(x):
    return 0.5 * x * (1.0 + jax.lax.erf(x * (2.0 ** -0.5)))


def _head_body(x_ref, W1_ref, W2_ref, b1_ref, b2_ref, rmsw_ref,
               We1_ref, be1_ref, We2_ref, be2_ref, cent_ref,
               feat_ref, d2T_ref, en_ref, nrm_ref,
               acc_ref, u1_ref, v1_ref, u2_ref, v2_ref, *, nsteps, L,
               n_iter=8):
    i = pl.program_id(0)

    @pl.when(i == 0)
    def _init():
        acc_ref[...] = jnp.zeros_like(acc_ref)
        u1_ref[...] = jnp.full_like(u1_ref, 1.0 / (u1_ref.shape[1] ** 0.5))
        u2_ref[...] = jnp.full_like(u2_ref, 1.0 / (u2_ref.shape[1] ** 0.5))

    @pl.when(i < n_iter)
    def _power_step():
        for W_ref, u_ref, v_ref in ((W1_ref, u1_ref, v1_ref),
                                    (W2_ref, u2_ref, v2_ref)):
            W = W_ref[...]
            v = _dot(u_ref[...], W, ((1,), (0,)))
            v = v / (jnp.sqrt(jnp.sum(v * v)) + 1e-12)
            u = _dot(v, W, ((1,), (1,)))
            u = u / (jnp.sqrt(jnp.sum(u * u)) + 1e-12)
            u_ref[...] = u
            v_ref[...] = v

    acc_ref[...] += jnp.sum(x_ref[...], axis=1)

    @pl.when(i == nsteps - 1)
    def _head():
        B = acc_ref.shape[0]
        pooled = acc_ref[...] * (1.0 / L)

        s1 = jnp.sum(u1_ref[...] * _dot(v1_ref[...], W1_ref[...],
                                        ((1,), (1,))))
        s2 = jnp.sum(u2_ref[...] * _dot(v2_ref[...], W2_ref[...],
                                        ((1,), (1,))))
        W1n = W1_ref[...] / s1
        W2n = W2_ref[...] / s2
        h1 = _gelu(_dot(pooled, W1n, ((1,), (1,))) + b1_ref[...])
        f_pre = _dot(h1, W2n, ((1,), (1,))) + b2_ref[...]
        rms = jax.lax.rsqrt(jnp.mean(f_pre * f_pre, axis=-1, keepdims=True)
                            + 1e-6)
        feat = f_pre * rms * rmsw_ref[...]
        feat_ref[...] = feat

        cent = cent_ref[...]
        cols = []
        for b in range(B):
            diff = cent - feat[b:b + 1, :]
            cols.append(jnp.sum(diff * diff, axis=1, keepdims=True))
        d2T_ref[...] = jnp.concatenate(cols, axis=1)

        be2s = jnp.sum(be2_ref[...])
        g1 = _gelu(_dot(feat, We1_ref[...], ((1,), (1,))) + be1_ref[...])
        g1b = g1.astype(jnp.bfloat16).astype(jnp.float32)
        We2 = We2_ref[...].astype(jnp.bfloat16).astype(jnp.float32)
        for b in range(B):
            en_b = jax.nn.sigmoid(jnp.sum(g1b[b:b + 1, :] * We2) + be2s)
            nrm_b = jnp.sqrt(jnp.sum(feat[b:b + 1, :] ** 2))
            en_ref[b:b + 1, :] = jnp.full((1, 1), en_b, jnp.float32)
            nrm_ref[b:b + 1, :] = jnp.full((1, 1), nrm_b, jnp.float32)


def _sc_assign(d2, iota, feat, cent):
    B, K = d2.shape
    H = cent.shape[1]
    NL = 16

    cp = pltpu.CompilerParams()
    if "needs_layout_passes" in pltpu.CompilerParams.__dataclass_fields__:
        cp = dataclasses.replace(cp, needs_layout_passes=False)

    @pl.kernel(
        out_type=jax.ShapeDtypeStruct((K, H), jnp.float32),
        mesh=plsc.VectorSubcoreMesh(core_axis_name="c", subcore_axis_name="s"),
        compiler_params=cp,
        scratch_types=[
            pltpu.VMEM((B, K), jnp.float32),
            pltpu.VMEM((1, NL), jnp.float32),
            pltpu.VMEM((B, H), jnp.float32),
            pltpu.VMEM((K, H), jnp.float32),
            pltpu.SemaphoreType.DMA,
        ],
    )
    def assign(d2_hbm, i_hbm, f_hbm, c_hbm, o_hbm, d2_v, io_v, f_v, c_v, sem):
        core = jax.lax.axis_index("c")
        sub = jax.lax.axis_index("s")

        @pl.when((core == 0) & (sub == 0))
        def _():
            pltpu.async_copy(d2_hbm, d2_v, sem).wait()
            pltpu.async_copy(i_hbm, io_v, sem).wait()
            pltpu.async_copy(f_hbm, f_v, sem).wait()
            pltpu.async_copy(c_hbm, c_v, sem).wait()

            iota16 = io_v[0, :]
            nearest = []
            for b in range(B):
                best = d2_v[b, pl.ds(0, NL)]
                bidx = iota16
                for c in range(1, K // NL):
                    chunk = d2_v[b, pl.ds(c * NL, NL)]
                    idx = iota16 + jnp.float32(c * NL)
                    better = chunk < best
                    best = jnp.where(better, chunk, best)
                    bidx = jnp.where(better, idx, bidx)
                m = jnp.min(best)
                cand = jnp.where(best == m, bidx, jnp.float32(K))
                nearest.append(jnp.min(cand))
            for b in range(B):
                kb = nearest[b].astype(jnp.int32)
                cnt = jnp.float32(0.0)
                for b2 in range(B):
                    cnt = cnt + jnp.where(nearest[b2] == nearest[b], 1.0, 0.0)
                rcnt = jnp.where(
                    cnt == 1.0, jnp.float32(1.0),
                    jnp.where(cnt == 2.0, jnp.float32(0.5),
                              jnp.where(cnt == 3.0, jnp.float32(1.0 / 3.0),
                                        jnp.float32(0.25))))
                w = (1.0 - _EMA) * rcnt
                for j in range(H // NL):
                    sl = pl.ds(j * NL, NL)
                    acc = jnp.zeros((NL,), jnp.float32)
                    for b2 in range(B):
                        m2 = jnp.where(nearest[b2] == nearest[b], 1.0, 0.0)
                        acc = acc + m2 * f_v[b2, sl]
                    c_v[kb, sl] = _EMA * c_v[kb, sl] + w * acc

            pltpu.async_copy(c_v, o_hbm, sem).wait()

    return assign(d2, iota, feat, cent)


def _scale_body(x_ref, cent_ref, feat_ref, prec_ref, en_ref, nrm_ref,
                WgT_ref, bg_ref,
                out_ref, ood_ref, mah_ref, eno_ref, su_ref, scale_ref):
    i = pl.program_id(0)

    @pl.when(i == 0)
    def _combine():
        B = feat_ref.shape[0]
        cent_new = cent_ref[...]
        feat = feat_ref[...]
        prec = prec_ref[...]
        mah_s, en_s, nrm_s = [], [], []
        for b in range(B):
            diff = cent_new - feat[b:b + 1, :]
            m = jnp.sum(diff * diff * prec, axis=1, keepdims=True)
            mah_s.append(jnp.sqrt(jnp.min(m)))
            en_s.append(jnp.sum(en_ref[b:b + 1, :]))
            nrm_s.append(jnp.sum(nrm_ref[b:b + 1, :]))
        mah_max = functools.reduce(jnp.maximum, mah_s)
        nrm_max = functools.reduce(jnp.maximum, nrm_s)
        WgT = WgT_ref[...]
        bg = bg_ref[...]
        for b in range(B):
            su_b = 1.0 - nrm_s[b] / (nrm_max + 1e-6)
            ood_b = (mah_s[b] / (mah_max + 1e-6) + en_s[b] + su_b) / 3.0
            gate = jax.nn.sigmoid(ood_b * WgT + bg)
            scale_ref[b:b + 1, :] = 0.7 + 0.3 * gate
            ood_ref[b:b + 1, :] = jnp.full((1, 1), ood_b, jnp.float32)
            mah_ref[b:b + 1, :] = jnp.full((1, 1), mah_s[b], jnp.float32)
            eno_ref[b:b + 1, :] = jnp.full((1, 1), en_s[b], jnp.float32)
            su_ref[b:b + 1, :] = jnp.full((1, 1), su_b, jnp.float32)

    out_ref[...] = x_ref[...] * scale_ref[...][:, None, :]


@jax.jit
def kernel(x, W1, b1, W2, b2, rms_w, We1, be1, We2, be2, Wg, bg,
           centroids, precision_diag):
    B, L, D = x.shape
    H = W1.shape[0]
    Hh = We1.shape[0]
    K = centroids.shape[0]
    LC = 256
    nsteps = L // LC

    full = lambda shape: pl.BlockSpec(shape, lambda i: (0,) * len(shape))

    feat, d2T, en, nrm = pl.pallas_call(
        functools.partial(_head_body, nsteps=nsteps, L=L),
        grid=(nsteps,),
        in_specs=[
            pl.BlockSpec((B, LC, D), lambda i: (0, i, 0)),
            full((H, D)), full((H, H)),
            full((1, H)), full((1, H)), full((1, H)),
            full((Hh, H)), full((1, Hh)), full((1, Hh)), full((1, 1)),
            full((K, H)),
        ],
        out_specs=[full((B, H)), full((K, B)),
                   full((B, 1)), full((B, 1))],
        out_shape=[
            jax.ShapeDtypeStruct((B, H), jnp.float32),
            jax.ShapeDtypeStruct((K, B), jnp.float32),
            jax.ShapeDtypeStruct((B, 1), jnp.float32),
            jax.ShapeDtypeStruct((B, 1), jnp.float32),
        ],
        scratch_shapes=[pltpu.VMEM((B, D), jnp.float32),
                        pltpu.VMEM((1, H), jnp.float32),
                        pltpu.VMEM((1, D), jnp.float32),
                        pltpu.VMEM((1, H), jnp.float32),
                        pltpu.VMEM((1, H), jnp.float32)],
    )(
        x, W1, W2,
        b1.reshape(1, H), b2.reshape(1, H), rms_w.reshape(1, H),
        We1, be1.reshape(1, Hh), We2, be2.reshape(1, 1),
        centroids,
    )

    iota = jnp.arange(16, dtype=jnp.float32).reshape(1, 16)
    cent_new = _sc_assign(d2T.T, iota, feat, centroids)

    x_ood, ood, mah, eno, su = pl.pallas_call(
        _scale_body,
        grid=(nsteps,),
        in_specs=[
            pl.BlockSpec((B, LC, D), lambda i: (0, i, 0)),
            full((K, H)), full((B, H)), full((1, H)),
            full((B, 1)), full((B, 1)),
            full((1, D)), full((1, D)),
        ],
        out_specs=[
            pl.BlockSpec((B, LC, D), lambda i: (0, i, 0)),
            full((B, 1)), full((B, 1)), full((B, 1)), full((B, 1)),
        ],
        out_shape=[
            jax.ShapeDtypeStruct((B, L, D), jnp.float32),
            jax.ShapeDtypeStruct((B, 1), jnp.float32),
            jax.ShapeDtypeStruct((B, 1), jnp.float32),
            jax.ShapeDtypeStruct((B, 1), jnp.float32),
            jax.ShapeDtypeStruct((B, 1), jnp.float32),
        ],
        scratch_shapes=[pltpu.VMEM((B, D), jnp.float32)],
        compiler_params=pltpu.CompilerParams(
            dimension_semantics=("arbitrary",)),
    )(x, cent_new, feat, precision_diag.reshape(1, H), en, nrm,
      Wg.reshape(1, D), bg.reshape(1, D))

    ood_score = ood.reshape(B)
    return (x_ood, ood_score, ood_score > _THRESHOLD, mah.reshape(B),
            eno.reshape(B), su.reshape(B))


# speedup vs baseline: 1.0295x; 1.0295x over previous
import dataclasses
import functools

import jax
import jax.numpy as jnp
from jax.experimental import pallas as pl
from jax.experimental.pallas import tpu as pltpu
from jax.experimental.pallas import tpu_sc as plsc

_EMA = 0.99
_THRESHOLD = 0.7


def _dot(a, b, dims):
    return jax.lax.dot_general(a, b, (dims, ((), ())),
                               precision=jax.lax.Precision.DEFAULT,
                               preferred_element_type=jnp.float32)


def _gelu(x):
    return 0.5 * x * (1.0 + jax.lax.erf(x * (2.0 ** -0.5)))


def _head_body(x_ref, W1_ref, W2_ref, b1_ref, b2_ref, rmsw_ref,
               We1_ref, be1_ref, We2_ref, be2_ref, cent_ref,
               feat_ref, d2T_ref, en_ref, nrm_ref,
               acc_ref, u1_ref, v1_ref, u2_ref, v2_ref, *, nsteps, L,
               n_iter=8):
    i = pl.program_id(0)

    @pl.when(i == 0)
    def _init():
        acc_ref[...] = jnp.zeros_like(acc_ref)
        u1_ref[...] = jnp.full_like(u1_ref, 1.0 / (u1_ref.shape[1] ** 0.5))
        u2_ref[...] = jnp.full_like(u2_ref, 1.0 / (u2_ref.shape[1] ** 0.5))

    @pl.when(i < n_iter)
    def _power_step():
        for W_ref, u_ref, v_ref in ((W1_ref, u1_ref, v1_ref),
                                    (W2_ref, u2_ref, v2_ref)):
            W = W_ref[...]
            v = _dot(u_ref[...], W, ((1,), (0,)))
            v = v / (jnp.sqrt(jnp.sum(v * v)) + 1e-12)
            u = _dot(v, W, ((1,), (1,)))
            u = u / (jnp.sqrt(jnp.sum(u * u)) + 1e-12)
            u_ref[...] = u
            v_ref[...] = v

    acc_ref[...] += jnp.sum(x_ref[...], axis=1)

    @pl.when(i == nsteps - 1)
    def _head():
        B = acc_ref.shape[0]
        pooled = acc_ref[...] * (1.0 / L)

        s1 = jnp.sum(u1_ref[...] * _dot(v1_ref[...], W1_ref[...],
                                        ((1,), (1,))))
        s2 = jnp.sum(u2_ref[...] * _dot(v2_ref[...], W2_ref[...],
                                        ((1,), (1,))))
        W1n = W1_ref[...] / s1
        W2n = W2_ref[...] / s2
        h1 = _gelu(_dot(pooled, W1n, ((1,), (1,))) + b1_ref[...])
        f_pre = _dot(h1, W2n, ((1,), (1,))) + b2_ref[...]
        rms = jax.lax.rsqrt(jnp.mean(f_pre * f_pre, axis=-1, keepdims=True)
                            + 1e-6)
        feat = f_pre * rms * rmsw_ref[...]
        feat_ref[...] = feat

        cent = cent_ref[...]
        cols = []
        for b in range(B):
            diff = cent - feat[b:b + 1, :]
            cols.append(jnp.sum(diff * diff, axis=1, keepdims=True))
        d2T_ref[...] = jnp.concatenate(cols, axis=1)

        be2s = jnp.sum(be2_ref[...])
        g1 = _gelu(_dot(feat, We1_ref[...], ((1,), (1,))) + be1_ref[...])
        g1b = g1.astype(jnp.bfloat16).astype(jnp.float32)
        We2 = We2_ref[...].astype(jnp.bfloat16).astype(jnp.float32)
        for b in range(B):
            en_b = jax.nn.sigmoid(jnp.sum(g1b[b:b + 1, :] * We2) + be2s)
            nrm_b = jnp.sqrt(jnp.sum(feat[b:b + 1, :] ** 2))
            en_ref[b:b + 1, :] = jnp.full((1, 1), en_b, jnp.float32)
            nrm_ref[b:b + 1, :] = jnp.full((1, 1), nrm_b, jnp.float32)


def _sc_assign(d2, iota, feat, cent):
    B, K = d2.shape
    H = cent.shape[1]
    NL = 16

    cp = pltpu.CompilerParams()
    if "needs_layout_passes" in pltpu.CompilerParams.__dataclass_fields__:
        cp = dataclasses.replace(cp, needs_layout_passes=False)

    @pl.kernel(
        out_type=jax.ShapeDtypeStruct((K, H), jnp.float32),
        mesh=plsc.VectorSubcoreMesh(core_axis_name="c", subcore_axis_name="s"),
        compiler_params=cp,
        scratch_types=[
            pltpu.VMEM((B, K), jnp.float32),
            pltpu.VMEM((1, NL), jnp.float32),
            pltpu.VMEM((B, H), jnp.float32),
            pltpu.VMEM((K, H), jnp.float32),
            pltpu.SemaphoreType.DMA,
            pltpu.SemaphoreType.DMA,
            pltpu.SemaphoreType.DMA,
            pltpu.SemaphoreType.DMA,
        ],
    )
    def assign(d2_hbm, i_hbm, f_hbm, c_hbm, o_hbm, d2_v, io_v, f_v, c_v,
               sem0, sem1, sem2, sem3):
        core = jax.lax.axis_index("c")
        sub = jax.lax.axis_index("s")

        @pl.when((core == 0) & (sub == 0))
        def _():
            sem = sem0
            cp0 = pltpu.async_copy(d2_hbm, d2_v, sem0)
            cp1 = pltpu.async_copy(i_hbm, io_v, sem1)
            cp2 = pltpu.async_copy(f_hbm, f_v, sem2)
            cp3 = pltpu.async_copy(c_hbm, c_v, sem3)
            cp0.wait()
            cp1.wait()
            cp2.wait()
            cp3.wait()

            iota16 = io_v[0, :]
            nearest = []
            for b in range(B):
                best = d2_v[b, pl.ds(0, NL)]
                bidx = iota16
                for c in range(1, K // NL):
                    chunk = d2_v[b, pl.ds(c * NL, NL)]
                    idx = iota16 + jnp.float32(c * NL)
                    better = chunk < best
                    best = jnp.where(better, chunk, best)
                    bidx = jnp.where(better, idx, bidx)
                m = jnp.min(best)
                cand = jnp.where(best == m, bidx, jnp.float32(K))
                nearest.append(jnp.min(cand))
            for b in range(B):
                kb = nearest[b].astype(jnp.int32)
                cnt = jnp.float32(0.0)
                for b2 in range(B):
                    cnt = cnt + jnp.where(nearest[b2] == nearest[b], 1.0, 0.0)
                rcnt = jnp.where(
                    cnt == 1.0, jnp.float32(1.0),
                    jnp.where(cnt == 2.0, jnp.float32(0.5),
                              jnp.where(cnt == 3.0, jnp.float32(1.0 / 3.0),
                                        jnp.float32(0.25))))
                w = (1.0 - _EMA) * rcnt
                for j in range(H // NL):
                    sl = pl.ds(j * NL, NL)
                    acc = jnp.zeros((NL,), jnp.float32)
                    for b2 in range(B):
                        m2 = jnp.where(nearest[b2] == nearest[b], 1.0, 0.0)
                        acc = acc + m2 * f_v[b2, sl]
                    c_v[kb, sl] = _EMA * c_v[kb, sl] + w * acc

            pltpu.async_copy(c_v, o_hbm, sem).wait()

    return assign(d2, iota, feat, cent)


def _scale_body(x_ref, cent_ref, feat_ref, prec_ref, en_ref, nrm_ref,
                WgT_ref, bg_ref,
                out_ref, ood_ref, mah_ref, eno_ref, su_ref, scale_ref):
    i = pl.program_id(0)

    @pl.when(i == 0)
    def _combine():
        B = feat_ref.shape[0]
        cent_new = cent_ref[...]
        feat = feat_ref[...]
        prec = prec_ref[...]
        mah_s, en_s, nrm_s = [], [], []
        for b in range(B):
            diff = cent_new - feat[b:b + 1, :]
            m = jnp.sum(diff * diff * prec, axis=1, keepdims=True)
            mah_s.append(jnp.sqrt(jnp.min(m)))
            en_s.append(jnp.sum(en_ref[b:b + 1, :]))
            nrm_s.append(jnp.sum(nrm_ref[b:b + 1, :]))
        mah_max = functools.reduce(jnp.maximum, mah_s)
        nrm_max = functools.reduce(jnp.maximum, nrm_s)
        WgT = WgT_ref[...]
        bg = bg_ref[...]
        for b in range(B):
            su_b = 1.0 - nrm_s[b] / (nrm_max + 1e-6)
            ood_b = (mah_s[b] / (mah_max + 1e-6) + en_s[b] + su_b) / 3.0
            gate = jax.nn.sigmoid(ood_b * WgT + bg)
            scale_ref[b:b + 1, :] = 0.7 + 0.3 * gate
            ood_ref[b:b + 1, :] = jnp.full((1, 1), ood_b, jnp.float32)
            mah_ref[b:b + 1, :] = jnp.full((1, 1), mah_s[b], jnp.float32)
            eno_ref[b:b + 1, :] = jnp.full((1, 1), en_s[b], jnp.float32)
            su_ref[b:b + 1, :] = jnp.full((1, 1), su_b, jnp.float32)

    out_ref[...] = x_ref[...] * scale_ref[...][:, None, :]


@jax.jit
def kernel(x, W1, b1, W2, b2, rms_w, We1, be1, We2, be2, Wg, bg,
           centroids, precision_diag):
    B, L, D = x.shape
    H = W1.shape[0]
    Hh = We1.shape[0]
    K = centroids.shape[0]
    LC = 256
    nsteps = L // LC

    full = lambda shape: pl.BlockSpec(shape, lambda i: (0,) * len(shape))

    feat, d2T, en, nrm = pl.pallas_call(
        functools.partial(_head_body, nsteps=nsteps, L=L),
        grid=(nsteps,),
        in_specs=[
            pl.BlockSpec((B, LC, D), lambda i: (0, i, 0)),
            full((H, D)), full((H, H)),
            full((1, H)), full((1, H)), full((1, H)),
            full((Hh, H)), full((1, Hh)), full((1, Hh)), full((1, 1)),
            full((K, H)),
        ],
        out_specs=[full((B, H)), full((K, B)),
                   full((B, 1)), full((B, 1))],
        out_shape=[
            jax.ShapeDtypeStruct((B, H), jnp.float32),
            jax.ShapeDtypeStruct((K, B), jnp.float32),
            jax.ShapeDtypeStruct((B, 1), jnp.float32),
            jax.ShapeDtypeStruct((B, 1), jnp.float32),
        ],
        scratch_shapes=[pltpu.VMEM((B, D), jnp.float32),
                        pltpu.VMEM((1, H), jnp.float32),
                        pltpu.VMEM((1, D), jnp.float32),
                        pltpu.VMEM((1, H), jnp.float32),
                        pltpu.VMEM((1, H), jnp.float32)],
    )(
        x, W1, W2,
        b1.reshape(1, H), b2.reshape(1, H), rms_w.reshape(1, H),
        We1, be1.reshape(1, Hh), We2, be2.reshape(1, 1),
        centroids,
    )

    iota = jnp.arange(16, dtype=jnp.float32).reshape(1, 16)
    cent_new = _sc_assign(d2T.T, iota, feat, centroids)

    x_ood, ood, mah, eno, su = pl.pallas_call(
        _scale_body,
        grid=(nsteps,),
        in_specs=[
            pl.BlockSpec((B, LC, D), lambda i: (0, i, 0)),
            full((K, H)), full((B, H)), full((1, H)),
            full((B, 1)), full((B, 1)),
            full((1, D)), full((1, D)),
        ],
        out_specs=[
            pl.BlockSpec((B, LC, D), lambda i: (0, i, 0)),
            full((B, 1)), full((B, 1)), full((B, 1)), full((B, 1)),
        ],
        out_shape=[
            jax.ShapeDtypeStruct((B, L, D), jnp.float32),
            jax.ShapeDtypeStruct((B, 1), jnp.float32),
            jax.ShapeDtypeStruct((B, 1), jnp.float32),
            jax.ShapeDtypeStruct((B, 1), jnp.float32),
            jax.ShapeDtypeStruct((B, 1), jnp.float32),
        ],
        scratch_shapes=[pltpu.VMEM((B, D), jnp.float32)],
        compiler_params=pltpu.CompilerParams(
            dimension_semantics=("arbitrary",)),
    )(x, cent_new, feat, precision_diag.reshape(1, H), en, nrm,
      Wg.reshape(1, D), bg.reshape(1, D))

    ood_score = ood.reshape(B)
    return (x_ood, ood_score, ood_score > _THRESHOLD, mah.reshape(B),
            eno.reshape(B), su.reshape(B))
